# Initial kernel scaffold; baseline (speedup 1.0000x reference)
import jax
import jax.numpy as jnp
from jax.experimental import pallas as pl


def kernel(x, atten):
    raise NotImplementedError("write your pallas kernel here")


# speedup vs baseline: 1.6215x; 1.6215x over previous
import jax
import jax.numpy as jnp
from jax.experimental import pallas as pl
from jax.experimental.pallas import tpu as pltpu

_HEADS = 12
_S = 2048
_D = 768
_OUT = 512
_NHEAD = 506
_TAIL_START = 507
_GROUP = 309
_R = 512
_NB = (_HEADS * _S) // _R


def _colsum_body(a_ref, out_ref, acc_ref):
    k = pl.program_id(1)
    blk = a_ref[0]
    off = (k * _R) % _S
    rows = jax.lax.broadcasted_iota(jnp.int32, (_R, _S), 0) + off
    cols = jax.lax.broadcasted_iota(jnp.int32, (_R, _S), 1)
    blk = jnp.where(rows == cols, 0.0, blk)
    part = blk.reshape(_R // 8, 8, _S).sum(axis=0) * (1.0 / _HEADS)

    @pl.when(k == 0)
    def _():
        acc_ref[...] = part

    @pl.when(k > 0)
    def _():
        acc_ref[...] = acc_ref[...] + part

    @pl.when(k == _NB - 1)
    def _():
        out_ref[...] = acc_ref[...].sum(axis=0).reshape(1, 1, _S)


def _select_body(sl_ref, ss_ref, x_ref, o_ref):
    sl = sl_ref[0]
    ss = ss_ref[0]
    jl = jax.lax.broadcasted_iota(jnp.int32, (1, _S), 1)
    isb = jax.lax.broadcasted_iota(jnp.int32, (_S, 1), 0)
    neg = jnp.float32(-jnp.inf)
    vl = jnp.where(jl == 0, neg, sl)
    vs = jnp.where(isb == 0, neg, ss)
    ii = jax.lax.broadcasted_iota(jnp.int32, (_S, _S), 0)
    jj = jax.lax.broadcasted_iota(jnp.int32, (_S, _S), 1)
    eq = vs == vl
    n_gt_lane = (vs > vl).astype(jnp.float32).sum(axis=0, keepdims=True)
    n_eq_lane = (eq & (ii < jj)).astype(jnp.float32).sum(axis=0, keepdims=True)
    rank_lane = n_gt_lane + n_eq_lane
    n_gt_sub = (vs < vl).astype(jnp.float32).sum(axis=1, keepdims=True)
    n_eq_sub = (eq & (jj < ii)).astype(jnp.float32).sum(axis=1, keepdims=True)
    rank_sub = n_gt_sub + n_eq_sub

    head_lane = (rank_lane < _NHEAD) | (jl == 0)
    tail_lane = (rank_lane >= _TAIL_START) & (jl > 0)
    head_sub = (rank_sub < _NHEAD) | (isb == 0)
    tail_sub = (rank_sub >= _TAIL_START) & (isb > 0)
    p_h = (head_sub & (ii < jj)).astype(jnp.float32).sum(axis=0, keepdims=True)
    p_t = (tail_sub & (ii < jj)).astype(jnp.float32).sum(axis=0, keepdims=True)
    c = ((p_t >= _GROUP).astype(jnp.float32)
         + (p_t >= 2 * _GROUP).astype(jnp.float32)
         + (p_t >= 3 * _GROUP).astype(jnp.float32)
         + (p_t >= 4 * _GROUP).astype(jnp.float32))
    r = jnp.where(head_lane, p_h,
                  jnp.where(tail_lane, float(_TAIL_START) + c, 1e9))
    w = jnp.where(tail_lane, 1.0 / _GROUP, 1.0)
    q = jax.lax.broadcasted_iota(jnp.int32, (_OUT, 1), 0).astype(jnp.float32)
    sel = jnp.where(q == r, w, 0.0)
    out = jax.lax.dot_general(sel, x_ref[0], (((1,), (0,)), ((), ())),
                              precision=jax.lax.Precision.HIGHEST,
                              preferred_element_type=jnp.float32)
    o_ref[...] = out.reshape(1, _OUT, _D)


def kernel(x, atten):
    B, S, D = x.shape
    a3 = atten.reshape(B, _HEADS * S, S)
    scores = pl.pallas_call(
        _colsum_body,
        grid=(B, _NB),
        in_specs=[pl.BlockSpec((1, _R, _S), lambda b, k: (b, k, 0))],
        out_specs=pl.BlockSpec((1, 1, _S), lambda b, k: (b, 0, 0)),
        out_shape=jax.ShapeDtypeStruct((B, 1, _S), jnp.float32),
        scratch_shapes=[pltpu.VMEM((8, _S), jnp.float32)],
    )(a3)
    s_sub = scores.reshape(B, _S, 1)
    out = pl.pallas_call(
        _select_body,
        grid=(B,),
        in_specs=[
            pl.BlockSpec((1, 1, _S), lambda b: (b, 0, 0)),
            pl.BlockSpec((1, _S, 1), lambda b: (b, 0, 0)),
            pl.BlockSpec((1, _S, _D), lambda b: (b, 0, 0)),
        ],
        out_specs=pl.BlockSpec((1, _OUT, _D), lambda b: (b, 0, 0)),
        out_shape=jax.ShapeDtypeStruct((B, _OUT, _D), jnp.float32),
    )(scores, s_sub, x)
    return out


# speedup vs baseline: 1.7143x; 1.0572x over previous
import jax
import jax.numpy as jnp
from jax.experimental import pallas as pl
from jax.experimental.pallas import tpu as pltpu

_HEADS = 12
_S = 2048
_D = 768
_OUT = 512
_NHEAD = 506
_TAIL_START = 507
_GROUP = 309
_R = 512
_NB = (_HEADS * _S) // _R


def _colsum_body(a_ref, out_ref, acc_ref, dacc_ref):
    k = pl.program_id(1)
    blk = a_ref[0]
    part = blk.reshape(_R // 8, 8, _S).sum(axis=0)

    @pl.when(k == 0)
    def _():
        acc_ref[...] = jnp.zeros((8, _S), jnp.float32)
        dacc_ref[...] = jnp.zeros((8, _S), jnp.float32)

    acc_ref[...] = acc_ref[...] + part

    rl = jax.lax.broadcasted_iota(jnp.int32, (_R, _R), 0)
    cl = jax.lax.broadcasted_iota(jnp.int32, (_R, _R), 1)
    for q in range(_S // _R):
        @pl.when(k % (_S // _R) == q)
        def _():
            sub = blk[:, q * _R:(q + 1) * _R]
            dv = jnp.where(rl == cl, sub, 0.0).reshape(_R // 8, 8, _R).sum(axis=0)
            dacc_ref[:, q * _R:(q + 1) * _R] = (
                dacc_ref[:, q * _R:(q + 1) * _R] + dv)

    @pl.when(k == _NB - 1)
    def _():
        out_ref[...] = ((acc_ref[...] - dacc_ref[...]).sum(axis=0)
                        * (1.0 / _HEADS)).reshape(1, 1, _S)


def _select_body(sl_ref, ss_ref, x_ref, o_ref):
    sl = sl_ref[0]
    ss = ss_ref[0]
    jl = jax.lax.broadcasted_iota(jnp.int32, (1, _S), 1)
    isb = jax.lax.broadcasted_iota(jnp.int32, (_S, 1), 0)
    neg = jnp.float32(-jnp.inf)
    vl = jnp.where(jl == 0, neg, sl)
    vs = jnp.where(isb == 0, neg, ss)
    ii = jax.lax.broadcasted_iota(jnp.int32, (_S, _S), 0)
    jj = jax.lax.broadcasted_iota(jnp.int32, (_S, _S), 1)
    eq = vs == vl
    n_gt_lane = (vs > vl).astype(jnp.float32).sum(axis=0, keepdims=True)
    n_eq_lane = (eq & (ii < jj)).astype(jnp.float32).sum(axis=0, keepdims=True)
    rank_lane = n_gt_lane + n_eq_lane
    n_gt_sub = (vs < vl).astype(jnp.float32).sum(axis=1, keepdims=True)
    n_eq_sub = (eq & (jj < ii)).astype(jnp.float32).sum(axis=1, keepdims=True)
    rank_sub = n_gt_sub + n_eq_sub

    head_lane = (rank_lane < _NHEAD) | (jl == 0)
    tail_lane = (rank_lane >= _TAIL_START) & (jl > 0)
    head_sub = (rank_sub < _NHEAD) | (isb == 0)
    tail_sub = (rank_sub >= _TAIL_START) & (isb > 0)
    p_h = (head_sub & (ii < jj)).astype(jnp.float32).sum(axis=0, keepdims=True)
    p_t = (tail_sub & (ii < jj)).astype(jnp.float32).sum(axis=0, keepdims=True)
    c = ((p_t >= _GROUP).astype(jnp.float32)
         + (p_t >= 2 * _GROUP).astype(jnp.float32)
         + (p_t >= 3 * _GROUP).astype(jnp.float32)
         + (p_t >= 4 * _GROUP).astype(jnp.float32))
    r = jnp.where(head_lane, p_h,
                  jnp.where(tail_lane, float(_TAIL_START) + c, 1e9))
    q = jax.lax.broadcasted_iota(jnp.int32, (_OUT, 1), 0).astype(jnp.float32)
    sel = jnp.where(q == r, 1.0, 0.0)
    out = jax.lax.dot_general(
        sel, x_ref[0], (((1,), (0,)), ((), ())),
        precision=jax.lax.Precision.HIGHEST,
        preferred_element_type=jnp.float32)
    out = jnp.where(q >= float(_TAIL_START), out * (1.0 / _GROUP), out)
    o_ref[...] = out.reshape(1, _OUT, _D)


def kernel(x, atten):
    B, S, D = x.shape
    a3 = atten.reshape(B, _HEADS * S, S)
    scores = pl.pallas_call(
        _colsum_body,
        grid=(B, _NB),
        in_specs=[pl.BlockSpec((1, _R, _S), lambda b, k: (b, k, 0))],
        out_specs=pl.BlockSpec((1, 1, _S), lambda b, k: (b, 0, 0)),
        out_shape=jax.ShapeDtypeStruct((B, 1, _S), jnp.float32),
        scratch_shapes=[pltpu.VMEM((8, _S), jnp.float32),
                        pltpu.VMEM((8, _S), jnp.float32)],
    )(a3)
    s_sub = scores.reshape(B, _S, 1)
    out = pl.pallas_call(
        _select_body,
        grid=(B,),
        in_specs=[
            pl.BlockSpec((1, 1, _S), lambda b: (b, 0, 0)),
            pl.BlockSpec((1, _S, 1), lambda b: (b, 0, 0)),
            pl.BlockSpec((1, _S, _D), lambda b: (b, 0, 0)),
        ],
        out_specs=pl.BlockSpec((1, _OUT, _D), lambda b: (b, 0, 0)),
        out_shape=jax.ShapeDtypeStruct((B, _OUT, _D), jnp.float32),
    )(scores, s_sub, x)
    return out
